# baseline (speedup 1.0000x reference)
import functools

import jax
import jax.numpy as jnp
from jax import lax
from jax.experimental import pallas as pl
from jax.experimental.pallas import tpu as pltpu
from jax.experimental.pallas import tpu_sc as plsc

_NUM_CONT = 13
_NUM_CAT = 26
_NUM_TOK = _NUM_CONT + _NUM_CAT
_NUM_BINS = 50
_VOCAB = 100000
_EMB = 64
_BATCH = 4096

_NC = 2
_NS = 16
_NW = _NC * _NS
_BPW = _BATCH // _NW


def _sc_gather_tokens(cont_t, cat_t, bin_pair, cat_pair):
    mesh = plsc.VectorSubcoreMesh(core_axis_name="c", subcore_axis_name="s")

    @functools.partial(
        pl.kernel,
        out_type=jax.ShapeDtypeStruct((_NUM_TOK * _EMB, _BATCH), jnp.float32),
        mesh=mesh,
        scratch_types=[
            pltpu.VMEM((_NUM_CONT, _BPW), jnp.float32),
            pltpu.VMEM((_NUM_CAT, _BPW), jnp.int32),
            pltpu.VMEM((_BPW,), jnp.int32),
            pltpu.VMEM((_BPW,), jnp.int32),
            pltpu.VMEM((_BPW, 2 * _EMB), jnp.float32),
            pltpu.VMEM((_EMB, _BPW), jnp.float32),
            pltpu.SemaphoreType.DMA,
        ],
        compiler_params=pltpu.CompilerParams(
            needs_layout_passes=False, use_tc_tiling_on_sc=True
        ),
    )
    def gather_kernel(cont_hbm, cat_hbm, bpair_hbm, cpair_hbm, out_hbm,
                      cv, qv, pidx_v, half_v, prow_v, tbuf_v, sem):
        wid = lax.axis_index("s") * _NC + lax.axis_index("c")
        base_b = wid * _BPW

        pltpu.sync_copy(cont_hbm.at[:, pl.ds(base_b, _BPW)], cv)
        pltpu.sync_copy(cat_hbm.at[:, pl.ds(base_b, _BPW)], qv)

        def move_block(fb, table_hbm):
            pltpu.async_copy(table_hbm.at[pidx_v], prow_v, sem).wait()

            def j_body(j, _):
                row16 = j * 16 + lax.iota(jnp.int32, 16)
                col_base = half_v[pl.ds(j * 16, 16)]

                def e_body(e, _):
                    tbuf_v[e, pl.ds(j * 16, 16)] = plsc.load_gather(
                        prow_v, [row16, col_base + e]
                    )
                    return 0

                lax.fori_loop(0, _EMB, e_body, 0)
                return 0

            lax.fori_loop(0, _BPW // 16, j_body, 0)
            pltpu.sync_copy(
                tbuf_v,
                out_hbm.at[pl.ds(fb * _EMB, _EMB), pl.ds(base_b, _BPW)],
            )

        def set_pair(j, r):
            pidx_v[pl.ds(j * 16, 16)] = lax.shift_right_logical(r, 1)
            half_v[pl.ds(j * 16, 16)] = lax.bitwise_and(r, 1) * _EMB

        def cont_body(f, _):
            def idx_body(j, _):
                x = cv[f, pl.ds(j * 16, 16)]
                t = jnp.clip((x * jnp.float32(_NUM_BINS)).astype(jnp.int32),
                             0, _NUM_BINS - 1)
                set_pair(j, t + f * _NUM_BINS)
                return 0

            lax.fori_loop(0, _BPW // 16, idx_body, 0)
            move_block(f, bpair_hbm)
            return 0

        lax.fori_loop(0, _NUM_CONT, cont_body, 0)

        def cat_body(f, _):
            def idx_body(j, _):
                set_pair(j, qv[f, pl.ds(j * 16, 16)] + f * _VOCAB)
                return 0

            lax.fori_loop(0, _BPW // 16, idx_body, 0)
            move_block(_NUM_CONT + f, cpair_hbm)
            return 0

        lax.fori_loop(0, _NUM_CAT, cat_body, 0)

    return gather_kernel(cont_t, cat_t, bin_pair, cat_pair)


def _project(tokens_t, W, b2):
    def body(tok_ref, w_ref, b_ref, out_ref):
        y = lax.dot_general(w_ref[...], tok_ref[...], (((1,), (0,)), ((), ())),
                            preferred_element_type=jnp.float32)
        out_ref[...] = (y + b_ref[...]).reshape(1, _EMB, _BATCH)

    return pl.pallas_call(
        body,
        grid=(_NUM_TOK,),
        in_specs=[
            pl.BlockSpec((_EMB, _BATCH), lambda f: (f, 0)),
            pl.BlockSpec((_EMB, _EMB), lambda f: (0, 0)),
            pl.BlockSpec((_EMB, 1), lambda f: (0, 0)),
        ],
        out_specs=pl.BlockSpec((1, _EMB, _BATCH), lambda f: (f, 0, 0)),
        out_shape=jax.ShapeDtypeStruct((_NUM_TOK, _EMB, _BATCH), jnp.float32),
    )(tokens_t, W, b2)


def kernel(continuous_features, categorical_features, bin_tables, cat_tables, W, b):
    tokens_t = _sc_gather_tokens(
        continuous_features.T,
        categorical_features.T.astype(jnp.int32),
        bin_tables.reshape(_NUM_CONT * _NUM_BINS // 2, 2 * _EMB),
        cat_tables.reshape(_NUM_CAT * _VOCAB // 2, 2 * _EMB),
    )
    out_t = _project(tokens_t, W, b.reshape(_EMB, 1))
    return jnp.transpose(out_t, (2, 0, 1))


# speedup vs baseline: 1.7937x; 1.7937x over previous
import functools

import jax
import jax.numpy as jnp
from jax import lax
from jax.experimental import pallas as pl
from jax.experimental.pallas import tpu as pltpu
from jax.experimental.pallas import tpu_sc as plsc

_NUM_CONT = 13
_NUM_CAT = 26
_NUM_TOK = _NUM_CONT + _NUM_CAT
_NUM_BINS = 50
_VOCAB = 100000
_EMB = 64
_BATCH = 4096

_NC = 2
_NS = 16
_NW = _NC * _NS
_BPW = _BATCH // _NW

_SPLIT = 49920
_VB = 4992
_NLO = _SPLIT // _VB
_NCH = _NLO + 1
_ZSTRIDE = _NCH * _VB


def _project_cat(xt, W, b2):
    def body(x0_ref, x1_ref, w_ref, b_ref, out_ref):
        def mm(x):
            return lax.dot_general(x, w_ref[...], (((0,), (1,)), ((), ())),
                                   preferred_element_type=jnp.float32)

        y = jnp.concatenate([mm(x0_ref[0]), mm(x1_ref[0])], axis=1)
        out_ref[...] = y + b_ref[...]

    return pl.pallas_call(
        body,
        grid=(_NUM_CAT, _NCH),
        in_specs=[
            pl.BlockSpec((1, _EMB, _VB),
                         lambda f, c: (f, 0, jnp.minimum(c, _NLO - 1))),
            pl.BlockSpec((1, _EMB, _VB), lambda f, c: (f, 0, c + _NLO)),
            pl.BlockSpec((_EMB, _EMB), lambda f, c: (0, 0)),
            pl.BlockSpec((1, 2 * _EMB), lambda f, c: (0, 0)),
        ],
        out_specs=pl.BlockSpec((_VB, 2 * _EMB),
                               lambda f, c: (f * _NCH + c, 0)),
        out_shape=jax.ShapeDtypeStruct((_NUM_CAT * _ZSTRIDE, 2 * _EMB),
                                       jnp.float32),
    )(xt, xt, W, b2)


def _project_bin(bin_flat, W, b2):
    def body(x_ref, w_ref, b_ref, out_ref):
        y = lax.dot_general(x_ref[...], w_ref[...], (((1,), (1,)), ((), ())),
                            preferred_element_type=jnp.float32)
        out_ref[...] = jnp.concatenate([y, y], axis=1) + b_ref[...]

    return pl.pallas_call(
        body,
        in_specs=[
            pl.BlockSpec((_NUM_CONT * _NUM_BINS, _EMB), lambda: (0, 0)),
            pl.BlockSpec((_EMB, _EMB), lambda: (0, 0)),
            pl.BlockSpec((1, 2 * _EMB), lambda: (0, 0)),
        ],
        out_specs=pl.BlockSpec((_NUM_CONT * _NUM_BINS, 2 * _EMB),
                               lambda: (0, 0)),
        out_shape=jax.ShapeDtypeStruct((_NUM_CONT * _NUM_BINS, 2 * _EMB),
                                       jnp.float32),
    )(bin_flat, W, b2)


def _sc_gather(cont_t, cat_t, zbin, zcat):
    mesh = plsc.VectorSubcoreMesh(core_axis_name="c", subcore_axis_name="s")

    @functools.partial(
        pl.kernel,
        out_type=jax.ShapeDtypeStruct((_NUM_TOK * _EMB, _BATCH), jnp.float32),
        mesh=mesh,
        scratch_types=[
            pltpu.VMEM((_NUM_CONT, _BPW), jnp.float32),
            pltpu.VMEM((_NUM_CAT, _BPW), jnp.int32),
            pltpu.VMEM((_BPW,), jnp.int32),
            pltpu.VMEM((_BPW,), jnp.int32),
            pltpu.VMEM((_BPW, 2 * _EMB), jnp.float32),
            pltpu.VMEM((_EMB, _BPW), jnp.float32),
            pltpu.SemaphoreType.DMA,
        ],
        compiler_params=pltpu.CompilerParams(
            needs_layout_passes=False, use_tc_tiling_on_sc=True
        ),
    )
    def gather_kernel(cont_hbm, cat_hbm, zbin_hbm, zcat_hbm, out_hbm,
                      cv, qv, pidx_v, half_v, prow_v, tbuf_v, sem):
        wid = lax.axis_index("s") * _NC + lax.axis_index("c")
        base_b = wid * _BPW

        pltpu.sync_copy(cont_hbm.at[:, pl.ds(base_b, _BPW)], cv)
        pltpu.sync_copy(cat_hbm.at[:, pl.ds(base_b, _BPW)], qv)

        def move_block(fb, table_hbm):
            pltpu.async_copy(table_hbm.at[pidx_v], prow_v, sem).wait()

            def j_body(j, _):
                row16 = j * 16 + lax.iota(jnp.int32, 16)
                col_base = half_v[pl.ds(j * 16, 16)]

                def e_body(e, _):
                    tbuf_v[e, pl.ds(j * 16, 16)] = plsc.load_gather(
                        prow_v, [row16, col_base + e]
                    )
                    return 0

                lax.fori_loop(0, _EMB, e_body, 0)
                return 0

            lax.fori_loop(0, _BPW // 16, j_body, 0)
            pltpu.sync_copy(
                tbuf_v,
                out_hbm.at[pl.ds(fb * _EMB, _EMB), pl.ds(base_b, _BPW)],
            )

        def cont_body(f, _):
            def idx_body(j, _):
                x = cv[f, pl.ds(j * 16, 16)]
                r = jnp.clip((x * jnp.float32(_NUM_BINS)).astype(jnp.int32),
                             0, _NUM_BINS - 1)
                pidx_v[pl.ds(j * 16, 16)] = r + f * _NUM_BINS
                half_v[pl.ds(j * 16, 16)] = jnp.zeros((16,), jnp.int32)
                return 0

            lax.fori_loop(0, _BPW // 16, idx_body, 0)
            move_block(f, zbin_hbm)
            return 0

        lax.fori_loop(0, _NUM_CONT, cont_body, 0)

        def cat_body(f, _):
            def idx_body(j, _):
                v = qv[f, pl.ds(j * 16, 16)]
                h = (v >= _SPLIT).astype(jnp.int32)
                pidx_v[pl.ds(j * 16, 16)] = v - h * _SPLIT + f * _ZSTRIDE
                half_v[pl.ds(j * 16, 16)] = h * _EMB
                return 0

            lax.fori_loop(0, _BPW // 16, idx_body, 0)
            move_block(_NUM_CONT + f, zcat_hbm)
            return 0

        lax.fori_loop(0, _NUM_CAT, cat_body, 0)

    return gather_kernel(cont_t, cat_t, zbin, zcat)


def kernel(continuous_features, categorical_features, bin_tables, cat_tables, W, b):
    b2 = jnp.concatenate([b, b]).reshape(1, 2 * _EMB)
    zcat = _project_cat(jnp.transpose(cat_tables, (0, 2, 1)), W, b2)
    zbin = _project_bin(bin_tables.reshape(_NUM_CONT * _NUM_BINS, _EMB), W, b2)
    out_t = _sc_gather(
        continuous_features.T,
        categorical_features.T.astype(jnp.int32),
        zbin,
        zcat,
    )
    return jnp.transpose(out_t.reshape(_NUM_TOK, _EMB, _BATCH), (2, 0, 1))


# speedup vs baseline: 1.9843x; 1.1063x over previous
import functools

import jax
import jax.numpy as jnp
from jax import lax
from jax.experimental import pallas as pl
from jax.experimental.pallas import tpu as pltpu
from jax.experimental.pallas import tpu_sc as plsc

_NUM_CONT = 13
_NUM_CAT = 26
_NUM_TOK = _NUM_CONT + _NUM_CAT
_NUM_BINS = 50
_VOCAB = 100000
_EMB = 64
_BATCH = 4096

_NC = 2
_NS = 16
_NW = _NC * _NS
_BPW = _BATCH // _NW

_SPLIT = 49920
_VB = 4992
_NLO = _SPLIT // _VB
_NCH = _NLO + 1
_ZSTRIDE = _NCH * _VB


def _project_cat(xt, W, b2):
    def body(x0_ref, x1_ref, w_ref, b_ref, out_ref):
        def mm(x):
            return lax.dot_general(x, w_ref[...], (((0,), (1,)), ((), ())),
                                   preferred_element_type=jnp.float32)

        y = jnp.concatenate([mm(x0_ref[0]), mm(x1_ref[0])], axis=1)
        out_ref[...] = y + b_ref[...]

    return pl.pallas_call(
        body,
        grid=(_NUM_CAT, _NCH),
        in_specs=[
            pl.BlockSpec((1, _EMB, _VB),
                         lambda f, c: (f, 0, jnp.minimum(c, _NLO - 1))),
            pl.BlockSpec((1, _EMB, _VB), lambda f, c: (f, 0, c + _NLO)),
            pl.BlockSpec((_EMB, _EMB), lambda f, c: (0, 0)),
            pl.BlockSpec((1, 2 * _EMB), lambda f, c: (0, 0)),
        ],
        out_specs=pl.BlockSpec((_VB, 2 * _EMB),
                               lambda f, c: (f * _NCH + c, 0)),
        out_shape=jax.ShapeDtypeStruct((_NUM_CAT * _ZSTRIDE, 2 * _EMB),
                                       jnp.float32),
    )(xt, xt, W, b2)


def _cont_tokens(cont_t, bin_tables, W, bcol):
    def body(x_ref, t_ref, w_ref, b_ref, out_ref):
        idx = jnp.clip((x_ref[0] * jnp.float32(_NUM_BINS)).astype(jnp.int32),
                       0, _NUM_BINS - 1)
        rows = lax.broadcasted_iota(jnp.int32, (_NUM_BINS, _BATCH), 0)
        onehot = (rows == idx).astype(jnp.float32)
        tok = lax.dot_general(t_ref[0], onehot, (((0,), (0,)), ((), ())),
                              preferred_element_type=jnp.float32)
        y = lax.dot_general(w_ref[...], tok, (((1,), (0,)), ((), ())),
                            preferred_element_type=jnp.float32)
        out_ref[...] = (y + b_ref[...]).reshape(1, _EMB, _BATCH)

    return pl.pallas_call(
        body,
        grid=(_NUM_CONT,),
        in_specs=[
            pl.BlockSpec((1, 1, _BATCH), lambda f: (f, 0, 0)),
            pl.BlockSpec((1, _NUM_BINS, _EMB), lambda f: (f, 0, 0)),
            pl.BlockSpec((_EMB, _EMB), lambda f: (0, 0)),
            pl.BlockSpec((_EMB, 1), lambda f: (0, 0)),
        ],
        out_specs=pl.BlockSpec((1, _EMB, _BATCH), lambda f: (f, 0, 0)),
        out_shape=jax.ShapeDtypeStruct((_NUM_CONT, _EMB, _BATCH), jnp.float32),
    )(cont_t, bin_tables, W, bcol)


def _sc_gather(cat_t, zcat):
    mesh = plsc.VectorSubcoreMesh(core_axis_name="c", subcore_axis_name="s")

    @functools.partial(
        pl.kernel,
        out_type=jax.ShapeDtypeStruct((_NUM_CAT * _EMB, _BATCH), jnp.float32),
        mesh=mesh,
        scratch_types=[
            pltpu.VMEM((_NUM_CAT, _BPW), jnp.int32),
            pltpu.VMEM((_BPW,), jnp.int32),
            pltpu.VMEM((_BPW,), jnp.int32),
            pltpu.VMEM((_BPW, 2 * _EMB), jnp.float32),
            pltpu.VMEM((_EMB, _BPW), jnp.float32),
            pltpu.SemaphoreType.DMA,
        ],
        compiler_params=pltpu.CompilerParams(
            needs_layout_passes=False, use_tc_tiling_on_sc=True
        ),
    )
    def gather_kernel(cat_hbm, zcat_hbm, out_hbm,
                      qv, pidx_v, half_v, prow_v, tbuf_v, sem):
        wid = lax.axis_index("s") * _NC + lax.axis_index("c")
        base_b = wid * _BPW

        pltpu.sync_copy(cat_hbm.at[:, pl.ds(base_b, _BPW)], qv)

        def move_block(fb, table_hbm):
            pltpu.async_copy(table_hbm.at[pidx_v], prow_v, sem).wait()

            def j_body(j, _):
                row16 = j * 16 + lax.iota(jnp.int32, 16)
                col_base = half_v[pl.ds(j * 16, 16)]

                def e_body(e, _):
                    tbuf_v[e, pl.ds(j * 16, 16)] = plsc.load_gather(
                        prow_v, [row16, col_base + e]
                    )
                    return 0

                lax.fori_loop(0, _EMB, e_body, 0)
                return 0

            lax.fori_loop(0, _BPW // 16, j_body, 0)
            pltpu.sync_copy(
                tbuf_v,
                out_hbm.at[pl.ds(fb * _EMB, _EMB), pl.ds(base_b, _BPW)],
            )

        def cat_body(f, _):
            def idx_body(j, _):
                v = qv[f, pl.ds(j * 16, 16)]
                h = (v >= _SPLIT).astype(jnp.int32)
                pidx_v[pl.ds(j * 16, 16)] = v - h * _SPLIT + f * _ZSTRIDE
                half_v[pl.ds(j * 16, 16)] = h * _EMB
                return 0

            lax.fori_loop(0, _BPW // 16, idx_body, 0)
            move_block(f, zcat_hbm)
            return 0

        lax.fori_loop(0, _NUM_CAT, cat_body, 0)

    return gather_kernel(cat_t, zcat)


def kernel(continuous_features, categorical_features, bin_tables, cat_tables, W, b):
    b2 = jnp.concatenate([b, b]).reshape(1, 2 * _EMB)
    zcat = _project_cat(jnp.transpose(cat_tables, (0, 2, 1)), W, b2)
    cat_t = _sc_gather(categorical_features.T.astype(jnp.int32), zcat)
    cont_t = _cont_tokens(
        continuous_features.T.reshape(_NUM_CONT, 1, _BATCH),
        bin_tables, W, b.reshape(_EMB, 1))
    out = jnp.concatenate(
        [cont_t, cat_t.reshape(_NUM_CAT, _EMB, _BATCH)], axis=0)
    return jnp.transpose(out, (2, 0, 1))


# speedup vs baseline: 2.2080x; 1.1127x over previous
import functools

import jax
import jax.numpy as jnp
from jax import lax
from jax.experimental import pallas as pl
from jax.experimental.pallas import tpu as pltpu
from jax.experimental.pallas import tpu_sc as plsc

_NUM_CONT = 13
_NUM_CAT = 26
_NUM_TOK = _NUM_CONT + _NUM_CAT
_NUM_BINS = 50
_VOCAB = 100000
_EMB = 64
_BATCH = 4096

_NC = 2
_NS = 16
_NW = _NC * _NS
_BPW = _BATCH // _NW

_SPLIT = 49920
_VB = 4992
_NLO = _SPLIT // _VB
_NCH = _NLO + 1
_ZSTRIDE = _NCH * _VB


def _project_cat(xt, W, b2, f0, nf):
    def body(x0_ref, x1_ref, w_ref, b_ref, out_ref):
        def mm(x):
            return lax.dot_general(x, w_ref[...], (((0,), (1,)), ((), ())),
                                   preferred_element_type=jnp.float32)

        y = jnp.concatenate([mm(x0_ref[0]), mm(x1_ref[0])], axis=1)
        out_ref[...] = y + b_ref[...]

    return pl.pallas_call(
        body,
        grid=(nf, _NCH),
        in_specs=[
            pl.BlockSpec((1, _EMB, _VB),
                         lambda f, c: (f0 + f, 0, jnp.minimum(c, _NLO - 1))),
            pl.BlockSpec((1, _EMB, _VB), lambda f, c: (f0 + f, 0, c + _NLO)),
            pl.BlockSpec((_EMB, _EMB), lambda f, c: (0, 0)),
            pl.BlockSpec((1, 2 * _EMB), lambda f, c: (0, 0)),
        ],
        out_specs=pl.BlockSpec((_VB, 2 * _EMB),
                               lambda f, c: (f * _NCH + c, 0)),
        out_shape=jax.ShapeDtypeStruct((nf * _ZSTRIDE, 2 * _EMB),
                                       jnp.float32),
    )(xt, xt, W, b2)


def _cont_tokens(cont_t, bin_tables, W, bcol):
    def body(x_ref, t_ref, w_ref, b_ref, out_ref):
        idx = jnp.clip((x_ref[0] * jnp.float32(_NUM_BINS)).astype(jnp.int32),
                       0, _NUM_BINS - 1)
        rows = lax.broadcasted_iota(jnp.int32, (_NUM_BINS, _BATCH), 0)
        onehot = (rows == idx).astype(jnp.float32)
        tok = lax.dot_general(t_ref[0], onehot, (((0,), (0,)), ((), ())),
                              preferred_element_type=jnp.float32)
        y = lax.dot_general(w_ref[...], tok, (((1,), (0,)), ((), ())),
                            preferred_element_type=jnp.float32)
        out_ref[...] = (y + b_ref[...]).reshape(1, _EMB, _BATCH)

    return pl.pallas_call(
        body,
        grid=(_NUM_CONT,),
        in_specs=[
            pl.BlockSpec((1, 1, _BATCH), lambda f: (f, 0, 0)),
            pl.BlockSpec((1, _NUM_BINS, _EMB), lambda f: (f, 0, 0)),
            pl.BlockSpec((_EMB, _EMB), lambda f: (0, 0)),
            pl.BlockSpec((_EMB, 1), lambda f: (0, 0)),
        ],
        out_specs=pl.BlockSpec((1, _EMB, _BATCH), lambda f: (f, 0, 0)),
        out_shape=jax.ShapeDtypeStruct((_NUM_CONT, _EMB, _BATCH), jnp.float32),
    )(cont_t, bin_tables, W, bcol)


def _sc_gather(cat_t, zcat, nf):
    mesh = plsc.VectorSubcoreMesh(core_axis_name="c", subcore_axis_name="s")

    @functools.partial(
        pl.kernel,
        out_type=jax.ShapeDtypeStruct((nf * _EMB, _BATCH), jnp.float32),
        mesh=mesh,
        scratch_types=[
            pltpu.VMEM((nf, _BPW), jnp.int32),
            pltpu.VMEM((_BPW,), jnp.int32),
            pltpu.VMEM((_BPW,), jnp.int32),
            pltpu.VMEM((_BPW, 2 * _EMB), jnp.float32),
            pltpu.VMEM((_EMB, _BPW), jnp.float32),
            pltpu.SemaphoreType.DMA,
        ],
        compiler_params=pltpu.CompilerParams(
            needs_layout_passes=False, use_tc_tiling_on_sc=True
        ),
    )
    def gather_kernel(cat_hbm, zcat_hbm, out_hbm,
                      qv, pidx_v, half_v, prow_v, tbuf_v, sem):
        wid = lax.axis_index("s") * _NC + lax.axis_index("c")
        base_b = wid * _BPW

        pltpu.sync_copy(cat_hbm.at[:, pl.ds(base_b, _BPW)], qv)

        def move_block(fb, table_hbm):
            pltpu.async_copy(table_hbm.at[pidx_v], prow_v, sem).wait()

            def j_body(j, _):
                row16 = j * 16 + lax.iota(jnp.int32, 16)
                col_base = half_v[pl.ds(j * 16, 16)]

                def e_body(e, _):
                    tbuf_v[e, pl.ds(j * 16, 16)] = plsc.load_gather(
                        prow_v, [row16, col_base + e]
                    )
                    return 0

                lax.fori_loop(0, _EMB, e_body, 0)
                return 0

            lax.fori_loop(0, _BPW // 16, j_body, 0)
            pltpu.sync_copy(
                tbuf_v,
                out_hbm.at[pl.ds(fb * _EMB, _EMB), pl.ds(base_b, _BPW)],
            )

        def cat_body(f, _):
            def idx_body(j, _):
                v = qv[f, pl.ds(j * 16, 16)]
                h = (v >= _SPLIT).astype(jnp.int32)
                pidx_v[pl.ds(j * 16, 16)] = v - h * _SPLIT + f * _ZSTRIDE
                half_v[pl.ds(j * 16, 16)] = h * _EMB
                return 0

            lax.fori_loop(0, _BPW // 16, idx_body, 0)
            move_block(f, zcat_hbm)
            return 0

        lax.fori_loop(0, nf, cat_body, 0)

    return gather_kernel(cat_t, zcat)


def kernel(continuous_features, categorical_features, bin_tables, cat_tables, W, b):
    b2 = jnp.concatenate([b, b]).reshape(1, 2 * _EMB)
    xt = jnp.transpose(cat_tables, (0, 2, 1))
    qt = categorical_features.T.astype(jnp.int32)
    half = _NUM_CAT // 2
    zcat0 = _project_cat(xt, W, b2, 0, half)
    g0 = _sc_gather(qt[:half], zcat0, half)
    zcat1 = _project_cat(xt, W, b2, half, _NUM_CAT - half)
    g1 = _sc_gather(qt[half:], zcat1, _NUM_CAT - half)
    cont_t = _cont_tokens(
        continuous_features.T.reshape(_NUM_CONT, 1, _BATCH),
        bin_tables, W, b.reshape(_EMB, 1))
    out = jnp.concatenate(
        [cont_t,
         g0.reshape(half, _EMB, _BATCH),
         g1.reshape(_NUM_CAT - half, _EMB, _BATCH)], axis=0)
    return jnp.transpose(out, (2, 0, 1))


# speedup vs baseline: 2.3927x; 1.0837x over previous
import functools

import jax
import jax.numpy as jnp
from jax import lax
from jax.experimental import pallas as pl
from jax.experimental.pallas import tpu as pltpu
from jax.experimental.pallas import tpu_sc as plsc

_NUM_CONT = 13
_NUM_CAT = 26
_NUM_TOK = _NUM_CONT + _NUM_CAT
_NUM_BINS = 50
_VOCAB = 100000
_EMB = 64
_BATCH = 4096

_NC = 2
_NS = 16
_NW = _NC * _NS
_BPW = _BATCH // _NW

_SPLIT = 49920
_VB = 4992
_NLO = _SPLIT // _VB
_NCH = _NLO + 1
_ZSTRIDE = _NCH * _VB


def _project_cat(xt, W, b2, f0, nf):
    def body(x0_ref, x1_ref, w_ref, b_ref, out_ref):
        def mm(x):
            return lax.dot_general(x, w_ref[...], (((0,), (1,)), ((), ())),
                                   preferred_element_type=jnp.float32)

        y = jnp.concatenate([mm(x0_ref[0]), mm(x1_ref[0])], axis=1)
        out_ref[...] = y + b_ref[...]

    return pl.pallas_call(
        body,
        grid=(nf, _NCH),
        in_specs=[
            pl.BlockSpec((1, _EMB, _VB),
                         lambda f, c: (f0 + f, 0, jnp.minimum(c, _NLO - 1))),
            pl.BlockSpec((1, _EMB, _VB), lambda f, c: (f0 + f, 0, c + _NLO)),
            pl.BlockSpec((_EMB, _EMB), lambda f, c: (0, 0)),
            pl.BlockSpec((1, 2 * _EMB), lambda f, c: (0, 0)),
        ],
        out_specs=pl.BlockSpec((_VB, 2 * _EMB),
                               lambda f, c: (f * _NCH + c, 0)),
        out_shape=jax.ShapeDtypeStruct((nf * _ZSTRIDE, 2 * _EMB),
                                       jnp.float32),
    )(xt, xt, W, b2)


def _cont_tokens(cont_t, bin_tables, W, bcol):
    def body(x_ref, t_ref, w_ref, b_ref, out_ref):
        idx = jnp.clip((x_ref[0] * jnp.float32(_NUM_BINS)).astype(jnp.int32),
                       0, _NUM_BINS - 1)
        rows = lax.broadcasted_iota(jnp.int32, (_NUM_BINS, _BATCH), 0)
        onehot = (rows == idx).astype(jnp.float32)
        tok = lax.dot_general(t_ref[0], onehot, (((0,), (0,)), ((), ())),
                              preferred_element_type=jnp.float32)
        y = lax.dot_general(w_ref[...], tok, (((1,), (0,)), ((), ())),
                            preferred_element_type=jnp.float32)
        out_ref[...] = (y + b_ref[...]).reshape(1, _EMB, _BATCH)

    return pl.pallas_call(
        body,
        grid=(_NUM_CONT,),
        in_specs=[
            pl.BlockSpec((1, 1, _BATCH), lambda f: (f, 0, 0)),
            pl.BlockSpec((1, _NUM_BINS, _EMB), lambda f: (f, 0, 0)),
            pl.BlockSpec((_EMB, _EMB), lambda f: (0, 0)),
            pl.BlockSpec((_EMB, 1), lambda f: (0, 0)),
        ],
        out_specs=pl.BlockSpec((1, _EMB, _BATCH), lambda f: (f, 0, 0)),
        out_shape=jax.ShapeDtypeStruct((_NUM_CONT, _EMB, _BATCH), jnp.float32),
    )(cont_t, bin_tables, W, bcol)


def _sc_gather(cat_t, zcat, nf):
    mesh = plsc.VectorSubcoreMesh(core_axis_name="c", subcore_axis_name="s")

    @functools.partial(
        pl.kernel,
        out_type=jax.ShapeDtypeStruct((nf * _EMB, _BATCH), jnp.float32),
        mesh=mesh,
        scratch_types=[
            pltpu.VMEM((nf, _BPW), jnp.int32),
            pltpu.VMEM((_BPW,), jnp.int32),
            pltpu.VMEM((_BPW,), jnp.int32),
            pltpu.VMEM((_BPW, 2 * _EMB), jnp.float32),
            pltpu.VMEM((_EMB, _BPW), jnp.float32),
            pltpu.SemaphoreType.DMA,
        ],
        compiler_params=pltpu.CompilerParams(
            needs_layout_passes=False, use_tc_tiling_on_sc=True
        ),
    )
    def gather_kernel(cat_hbm, zcat_hbm, out_hbm,
                      qv, pidx_v, half_v, prow_v, tbuf_v, sem):
        wid = lax.axis_index("s") * _NC + lax.axis_index("c")
        base_b = wid * _BPW

        pltpu.sync_copy(cat_hbm.at[:, pl.ds(base_b, _BPW)], qv)

        def move_block(fb, table_hbm):
            pltpu.async_copy(table_hbm.at[pidx_v], prow_v, sem).wait()

            def j_body(j, _):
                row16 = j * 16 + lax.iota(jnp.int32, 16)
                col_base = half_v[pl.ds(j * 16, 16)]

                def e_body(e, _):
                    tbuf_v[e, pl.ds(j * 16, 16)] = plsc.load_gather(
                        prow_v, [row16, col_base + e]
                    )
                    return 0

                lax.fori_loop(0, _EMB, e_body, 0)
                return 0

            lax.fori_loop(0, _BPW // 16, j_body, 0)
            pltpu.sync_copy(
                tbuf_v,
                out_hbm.at[pl.ds(fb * _EMB, _EMB), pl.ds(base_b, _BPW)],
            )

        def cat_body(f, _):
            def idx_body(j, _):
                v = qv[f, pl.ds(j * 16, 16)]
                h = (v >= _SPLIT).astype(jnp.int32)
                pidx_v[pl.ds(j * 16, 16)] = v - h * _SPLIT + f * _ZSTRIDE
                half_v[pl.ds(j * 16, 16)] = h * _EMB
                return 0

            lax.fori_loop(0, _BPW // 16, idx_body, 0)
            move_block(f, zcat_hbm)
            return 0

        lax.fori_loop(0, nf, cat_body, 0)

    return gather_kernel(cat_t, zcat)


def kernel(continuous_features, categorical_features, bin_tables, cat_tables, W, b):
    b2 = jnp.concatenate([b, b]).reshape(1, 2 * _EMB)
    xt = jnp.transpose(cat_tables, (0, 2, 1))
    qt = categorical_features.T.astype(jnp.int32)
    gathered = []
    for f0, nf in ((0, 7), (7, 7), (14, 6), (20, 6)):
        zc = _project_cat(xt, W, b2, f0, nf)
        g = _sc_gather(qt[f0:f0 + nf], zc, nf)
        gathered.append(g.reshape(nf, _EMB, _BATCH))
    cont_t = _cont_tokens(
        continuous_features.T.reshape(_NUM_CONT, 1, _BATCH),
        bin_tables, W, b.reshape(_EMB, 1))
    out = jnp.concatenate([cont_t] + gathered, axis=0)
    return jnp.transpose(out, (2, 0, 1))
